# baseline (speedup 1.0000x reference)
import functools

import jax
import jax.numpy as jnp
from jax import lax
from jax.experimental import pallas as pl
from jax.experimental.pallas import tpu as pltpu
from jax.experimental.pallas import tpu_sc as plsc

_N = 10000
_E = 320000
_D = 128
_H = 128
_L = 3

_NC = 2
_NS = 16
_NW = _NC * _NS
_EPW = _E // _NW
_CHUNK = 128
_STEPS = 80
_EPWP = _STEPS * _CHUNK
_NP = 10240
_RPS = _NP // _NS
_NBUF = 1


def _sc_partials_body(h_hbm, src_hbm, dst_hbm, out_hbm,
                      sidx, dring, bufs, acc, gsems, ssems, isems):
    cid = lax.axis_index("c")
    sid = lax.axis_index("s")
    wid = sid * _NC + cid
    row0 = sid * _RPS

    def idx_start(c, b):
        pltpu.async_copy(dst_hbm.at[wid, c], dring.at[b], isems.at[b])

    def idx_wait(c, b):
        pltpu.make_async_copy(dst_hbm.at[wid, c], dring.at[b],
                              isems.at[b]).wait()

    def gather_start(c, b):
        pltpu.async_copy(h_hbm.at[sidx.at[pl.ds(c * _CHUNK, _CHUNK)]],
                         bufs.at[b], gsems.at[b])

    def gather_wait(c, b):
        pltpu.make_async_copy(h_hbm.at[sidx.at[pl.ds(c * _CHUNK, _CHUNK)]],
                              bufs.at[b], gsems.at[b]).wait()

    def scatter_start(b):
        pltpu.async_copy(bufs.at[b], acc.at[dring.at[b, 0]], ssems.at[b],
                         add=True)

    def scatter_wait(b):
        pltpu.make_async_copy(bufs.at[b], acc.at[dring.at[b, 0]],
                              ssems.at[b]).wait()

    pltpu.sync_copy(src_hbm.at[pl.ds(wid * _EPWP, _EPWP)], sidx)

    @pl.loop(0, _CHUNK)
    def _zr(r):
        @pl.loop(0, _H, step=16)
        def _zc(c):
            bufs[0, r, pl.ds(c, 16)] = jnp.zeros((16,), jnp.float32)

    @pl.loop(0, _RPS, step=_CHUNK)
    def _za(r):
        pltpu.sync_copy(bufs.at[0], acc.at[pl.ds(row0 + r, _CHUNK)])

    plsc.subcore_barrier()

    for b in range(_NBUF):
        idx_start(b, b)
        gather_start(b, b)

    @pl.loop(0, _STEPS - _NBUF, step=_NBUF)
    def _main(s):
        for b in range(_NBUF):
            gather_wait(s + b, b)
            idx_wait(s + b, b)
            scatter_start(b)
        for b in range(_NBUF):
            scatter_wait(b)
            idx_start(s + _NBUF + b, b)
            gather_start(s + _NBUF + b, b)

    for b in range(_NBUF):
        gather_wait(_STEPS - _NBUF + b, b)
        idx_wait(_STEPS - _NBUF + b, b)
        scatter_start(b)
    for b in range(_NBUF):
        scatter_wait(b)

    plsc.subcore_barrier()

    pltpu.sync_copy(acc.at[pl.ds(row0, _RPS)],
                    out_hbm.at[cid, pl.ds(row0, _RPS)])


@jax.jit
def _sc_partials(h, src1, dst4):
    kern = pl.kernel(
        _sc_partials_body,
        out_type=jax.ShapeDtypeStruct((_NC, _NP, _H), jnp.float32),
        mesh=plsc.VectorSubcoreMesh(core_axis_name="c", subcore_axis_name="s"),
        scratch_types=[
            pltpu.VMEM((_EPWP,), jnp.int32),
            pltpu.VMEM((_NBUF, 1, _CHUNK), jnp.int32),
            pltpu.VMEM((_NBUF, _CHUNK, _H), jnp.float32),
            pltpu.VMEM_SHARED((_NP, _H), jnp.float32),
            pltpu.SemaphoreType.DMA((_NBUF,)),
            pltpu.SemaphoreType.DMA((_NBUF,)),
            pltpu.SemaphoreType.DMA((_NBUF,)),
        ],
    )
    return kern(h, src1, dst4)


def _enc_body(x_ref, w_ref, b_ref, o_ref):
    o_ref[...] = (
        jnp.dot(x_ref[...], w_ref[...], preferred_element_type=jnp.float32)
        + b_ref[...]
    )


@jax.jit
def _encode(x, Wenc, benc):
    return pl.pallas_call(
        _enc_body,
        out_shape=jax.ShapeDtypeStruct((_N, _H), jnp.float32),
    )(x, Wenc, benc.reshape(1, _H))


def _bn_relu(z, g, b):
    m = jnp.mean(z, axis=0, keepdims=True)
    v = jnp.mean(jnp.square(z - m), axis=0, keepdims=True)
    z = g * (z - m) / jnp.sqrt(v + 1e-5) + b
    return jnp.maximum(z, 0.0)


def _gin_mlp(h_ref, p_ref, w1_ref, b1_ref, gm_ref, bm_ref,
             w2_ref, b2_ref, go_ref, bo_ref, sc_ref):
    h = h_ref[...]
    z = sc_ref[...] * h + (p_ref[0, : _N] + p_ref[1, : _N])
    z = jnp.dot(z, w1_ref[...], preferred_element_type=jnp.float32) + b1_ref[...]
    z = _bn_relu(z, gm_ref[...], bm_ref[...])
    z = jnp.dot(z, w2_ref[...], preferred_element_type=jnp.float32) + b2_ref[...]
    z = _bn_relu(z, go_ref[...], bo_ref[...])
    return z + h


def _layer_body(h_ref, p_ref, w1_ref, b1_ref, gm_ref, bm_ref,
                w2_ref, b2_ref, go_ref, bo_ref, sc_ref, o_ref):
    o_ref[...] = _gin_mlp(h_ref, p_ref, w1_ref, b1_ref, gm_ref, bm_ref,
                          w2_ref, b2_ref, go_ref, bo_ref, sc_ref)


def _last_body(h_ref, p_ref, w1_ref, b1_ref, gm_ref, bm_ref,
               w2_ref, b2_ref, go_ref, bo_ref, sc_ref,
               wo_ref, bo2_ref, o_ref):
    hn = _gin_mlp(h_ref, p_ref, w1_ref, b1_ref, gm_ref, bm_ref,
                  w2_ref, b2_ref, go_ref, bo_ref, sc_ref)
    o_ref[...] = (
        jnp.dot(hn, wo_ref[...], preferred_element_type=jnp.float32)
        + bo2_ref[...]
    )


@jax.jit
def _layer(*args):
    return pl.pallas_call(
        _layer_body,
        out_shape=jax.ShapeDtypeStruct((_N, _H), jnp.float32),
    )(*args)


@jax.jit
def _last(*args):
    return pl.pallas_call(
        _last_body,
        out_shape=jax.ShapeDtypeStruct((_N, 1), jnp.float32),
    )(*args)


def kernel(x, edge_index, Wenc, benc, W1, b1, g_mid, bt_mid, W2, b2,
           eps, g_out, bt_out, Wout, bout):
    pad = _EPWP - _EPW
    src1 = jnp.concatenate(
        [edge_index[0].astype(jnp.int32).reshape(_NW, _EPW),
         jnp.zeros((_NW, pad), jnp.int32)], axis=1).reshape(_NW * _EPWP)
    dst4 = jnp.concatenate(
        [edge_index[1].astype(jnp.int32).reshape(_NW, _EPW),
         jnp.full((_NW, pad), _N, jnp.int32)], axis=1
    ).reshape(_NW, _STEPS, 1, _CHUNK)
    h = _encode(x, Wenc, benc)
    for i in range(_L):
        parts = _sc_partials(h, src1, dst4)
        sc = (1.0 + eps[i]) * jnp.ones((1, _H), jnp.float32)
        args = (h, parts, W1[i], b1[i].reshape(1, -1),
                g_mid[i].reshape(1, -1), bt_mid[i].reshape(1, -1),
                W2[i], b2[i].reshape(1, -1),
                g_out[i].reshape(1, -1), bt_out[i].reshape(1, -1), sc)
        if i < _L - 1:
            h = _layer(*args)
        else:
            out = _last(*args, Wout, bout.reshape(1, 1))
    return out


# speedup vs baseline: 1.1235x; 1.1235x over previous
import functools

import jax
import jax.numpy as jnp
from jax import lax
from jax.experimental import pallas as pl
from jax.experimental.pallas import tpu as pltpu
from jax.experimental.pallas import tpu_sc as plsc

_N = 10000
_E = 320000
_D = 128
_H = 128
_L = 3

_NC = 2
_NS = 16
_NW = _NC * _NS
_EPW = _E // _NW
_CHUNK = 80
_STEPS = 128
_EPWP = _STEPS * _CHUNK
_NP = 10240
_RPS = _NP // _NS
_NSLOT = 2
_NIDX = 4


def _sc_partials_body(h_hbm, src_hbm, dst_hbm, out_hbm,
                      sidx, didx, bufs, acc, gsems, ssems, isS, isD):
    cid = lax.axis_index("c")
    sid = lax.axis_index("s")
    wid = sid * _NC + cid
    row0 = sid * _RPS
    ebase = wid * _EPWP

    def idx_start(c, k):
        off = ebase + c * _CHUNK
        pltpu.async_copy(src_hbm.at[pl.ds(off, _CHUNK)], sidx.at[k, 0],
                         isS.at[k])
        pltpu.async_copy(dst_hbm.at[pl.ds(off, _CHUNK)], didx.at[k, 0],
                         isD.at[k])

    def sidx_wait(c, k):
        off = ebase + c * _CHUNK
        pltpu.make_async_copy(src_hbm.at[pl.ds(off, _CHUNK)], sidx.at[k, 0],
                              isS.at[k]).wait()

    def didx_wait(c, k):
        off = ebase + c * _CHUNK
        pltpu.make_async_copy(dst_hbm.at[pl.ds(off, _CHUNK)], didx.at[k, 0],
                              isD.at[k]).wait()

    def gather_start(b, k):
        pltpu.async_copy(h_hbm.at[sidx.at[k, 0]], bufs.at[b], gsems.at[b])

    def gather_wait(b, k):
        pltpu.make_async_copy(h_hbm.at[sidx.at[k, 0]], bufs.at[b],
                              gsems.at[b]).wait()

    def scatter_start(b, k):
        pltpu.async_copy(bufs.at[b], acc.at[didx.at[k, 0]], ssems.at[b],
                         add=True)

    def scatter_wait(b, k):
        pltpu.make_async_copy(bufs.at[b], acc.at[didx.at[k, 0]],
                              ssems.at[b]).wait()

    @pl.loop(0, _CHUNK)
    def _zr(r):
        @pl.loop(0, _H, step=16)
        def _zc(c):
            bufs[0, r, pl.ds(c, 16)] = jnp.zeros((16,), jnp.float32)

    @pl.loop(0, _RPS, step=_CHUNK)
    def _za(r):
        pltpu.sync_copy(bufs.at[0], acc.at[pl.ds(row0 + r, _CHUNK)])

    plsc.subcore_barrier()

    for k in range(_NIDX):
        idx_start(k, k)
    for k in range(_NSLOT):
        sidx_wait(k, k)
        gather_start(k, k)

    @pl.loop(0, _STEPS - _NIDX, step=_NIDX)
    def _main(s):
        for k in range(_NIDX):
            c = s + k
            b = k % _NSLOT
            gather_wait(b, k)
            didx_wait(c, k)
            scatter_start(b, k)
            scatter_wait(b, k)
            idx_start(c + _NIDX, k)
            sidx_wait(c + _NSLOT, (k + _NSLOT) % _NIDX)
            gather_start(b, (k + _NSLOT) % _NIDX)

    for k in range(_NIDX):
        c = _STEPS - _NIDX + k
        b = k % _NSLOT
        gather_wait(b, k)
        didx_wait(c, k)
        scatter_start(b, k)
        scatter_wait(b, k)
        if k < _NSLOT:
            sidx_wait(c + _NSLOT, (k + _NSLOT) % _NIDX)
            gather_start(b, (k + _NSLOT) % _NIDX)

    plsc.subcore_barrier()

    pltpu.sync_copy(acc.at[pl.ds(row0, _RPS)],
                    out_hbm.at[cid, pl.ds(row0, _RPS)])


@jax.jit
def _sc_partials(h, src1, dst1):
    kern = pl.kernel(
        _sc_partials_body,
        out_type=jax.ShapeDtypeStruct((_NC, _NP, _H), jnp.float32),
        mesh=plsc.VectorSubcoreMesh(core_axis_name="c", subcore_axis_name="s"),
        scratch_types=[
            pltpu.VMEM((_NIDX, 1, _CHUNK), jnp.int32),
            pltpu.VMEM((_NIDX, 1, _CHUNK), jnp.int32),
            pltpu.VMEM((_NSLOT, _CHUNK, _H), jnp.float32),
            pltpu.VMEM_SHARED((_NP, _H), jnp.float32),
            pltpu.SemaphoreType.DMA((_NSLOT,)),
            pltpu.SemaphoreType.DMA((_NSLOT,)),
            pltpu.SemaphoreType.DMA((_NIDX,)),
            pltpu.SemaphoreType.DMA((_NIDX,)),
        ],
    )
    return kern(h, src1, dst1)


def _enc_body(x_ref, w_ref, b_ref, o_ref):
    o_ref[...] = (
        jnp.dot(x_ref[...], w_ref[...], preferred_element_type=jnp.float32)
        + b_ref[...]
    )


@jax.jit
def _encode(x, Wenc, benc):
    return pl.pallas_call(
        _enc_body,
        out_shape=jax.ShapeDtypeStruct((_N, _H), jnp.float32),
    )(x, Wenc, benc.reshape(1, _H))


def _bn_relu(z, g, b):
    m = jnp.mean(z, axis=0, keepdims=True)
    v = jnp.mean(jnp.square(z - m), axis=0, keepdims=True)
    z = g * (z - m) / jnp.sqrt(v + 1e-5) + b
    return jnp.maximum(z, 0.0)


def _gin_mlp(h_ref, p_ref, w1_ref, b1_ref, gm_ref, bm_ref,
             w2_ref, b2_ref, go_ref, bo_ref, sc_ref):
    h = h_ref[...]
    z = sc_ref[...] * h + (p_ref[0, : _N] + p_ref[1, : _N])
    z = jnp.dot(z, w1_ref[...], preferred_element_type=jnp.float32) + b1_ref[...]
    z = _bn_relu(z, gm_ref[...], bm_ref[...])
    z = jnp.dot(z, w2_ref[...], preferred_element_type=jnp.float32) + b2_ref[...]
    z = _bn_relu(z, go_ref[...], bo_ref[...])
    return z + h


def _layer_body(h_ref, p_ref, w1_ref, b1_ref, gm_ref, bm_ref,
                w2_ref, b2_ref, go_ref, bo_ref, sc_ref, o_ref):
    o_ref[...] = _gin_mlp(h_ref, p_ref, w1_ref, b1_ref, gm_ref, bm_ref,
                          w2_ref, b2_ref, go_ref, bo_ref, sc_ref)


def _last_body(h_ref, p_ref, w1_ref, b1_ref, gm_ref, bm_ref,
               w2_ref, b2_ref, go_ref, bo_ref, sc_ref,
               wo_ref, bo2_ref, o_ref):
    hn = _gin_mlp(h_ref, p_ref, w1_ref, b1_ref, gm_ref, bm_ref,
                  w2_ref, b2_ref, go_ref, bo_ref, sc_ref)
    o_ref[...] = (
        jnp.dot(hn, wo_ref[...], preferred_element_type=jnp.float32)
        + bo2_ref[...]
    )


@jax.jit
def _layer(*args):
    return pl.pallas_call(
        _layer_body,
        out_shape=jax.ShapeDtypeStruct((_N, _H), jnp.float32),
    )(*args)


@jax.jit
def _last(*args):
    return pl.pallas_call(
        _last_body,
        out_shape=jax.ShapeDtypeStruct((_N, 1), jnp.float32),
    )(*args)


def kernel(x, edge_index, Wenc, benc, W1, b1, g_mid, bt_mid, W2, b2,
           eps, g_out, bt_out, Wout, bout):
    pad = _EPWP - _EPW
    src1 = jnp.concatenate(
        [edge_index[0].astype(jnp.int32).reshape(_NW, _EPW),
         jnp.zeros((_NW, pad), jnp.int32)], axis=1).reshape(_NW * _EPWP)
    dst1 = jnp.concatenate(
        [edge_index[1].astype(jnp.int32).reshape(_NW, _EPW),
         jnp.full((_NW, pad), _N, jnp.int32)], axis=1).reshape(_NW * _EPWP)
    h = _encode(x, Wenc, benc)
    for i in range(_L):
        parts = _sc_partials(h, src1, dst1)
        sc = (1.0 + eps[i]) * jnp.ones((1, _H), jnp.float32)
        args = (h, parts, W1[i], b1[i].reshape(1, -1),
                g_mid[i].reshape(1, -1), bt_mid[i].reshape(1, -1),
                W2[i], b2[i].reshape(1, -1),
                g_out[i].reshape(1, -1), bt_out[i].reshape(1, -1), sc)
        if i < _L - 1:
            h = _layer(*args)
        else:
            out = _last(*args, Wout, bout.reshape(1, 1))
    return out
